# baseline (speedup 1.0000x reference)
import functools

import jax
import jax.numpy as jnp
from jax import lax
from jax.experimental import pallas as pl
from jax.experimental.pallas import tpu as pltpu
from jax.experimental.pallas import tpu_sc as plsc

B = 16384
EMB = 16
HID = 32
NC = 2
NS = 16
NW = NC * NS
BPW = B // NW
CH = 128
NCH = BPW // CH


def _sc_gather(user, quiz, user_table, quiz_table):
    mesh = plsc.VectorSubcoreMesh(core_axis_name="c", subcore_axis_name="s")

    @functools.partial(
        pl.kernel,
        mesh=mesh,
        out_type=[
            jax.ShapeDtypeStruct((B, EMB), jnp.float32),
            jax.ShapeDtypeStruct((B, EMB), jnp.float32),
        ],
        scratch_types=[
            pltpu.VMEM((NCH, CH), jnp.int32),
            pltpu.VMEM((NCH, CH), jnp.int32),
            pltpu.VMEM((BPW, EMB), jnp.float32),
            pltpu.VMEM((BPW, EMB), jnp.float32),
            pltpu.SemaphoreType.DMA,
            pltpu.SemaphoreType.DMA,
        ],
        compiler_params=pltpu.CompilerParams(use_tc_tiling_on_sc=False),
    )
    def k(user_hbm, quiz_hbm, utab_hbm, qtab_hbm, uout_hbm, qout_hbm,
          uidx_v, qidx_v, urows_v, qrows_v, usem, qsem):
        wid = lax.axis_index("s") * NC + lax.axis_index("c")
        base = wid * BPW
        for j in range(NCH):
            pltpu.sync_copy(user_hbm.at[pl.ds(base + j * CH, CH)], uidx_v.at[j])
            pltpu.sync_copy(quiz_hbm.at[pl.ds(base + j * CH, CH)], qidx_v.at[j])
        copies = []
        for j in range(NCH):
            copies.append(pltpu.async_copy(
                utab_hbm.at[uidx_v.at[j]], urows_v.at[pl.ds(j * CH, CH)], usem))
            copies.append(pltpu.async_copy(
                qtab_hbm.at[qidx_v.at[j]], qrows_v.at[pl.ds(j * CH, CH)], qsem))
        for c in copies:
            c.wait()
        pltpu.sync_copy(urows_v, uout_hbm.at[pl.ds(base, BPW)])
        pltpu.sync_copy(qrows_v, qout_hbm.at[pl.ds(base, BPW)])

    return k(user, quiz, user_table, quiz_table)


def _mlp_body(u_ref, q_ref, t_ref, w1u_ref, w1q_ref, w1t_ref, b1_ref,
              w2_ref, b2_ref, o_ref):
    x = (jnp.dot(u_ref[...], w1u_ref[...], preferred_element_type=jnp.float32)
         + jnp.dot(q_ref[...], w1q_ref[...], preferred_element_type=jnp.float32)
         + t_ref[...] * w1t_ref[...]
         + b1_ref[...])
    h = jnp.maximum(x, 0.0)
    z = jnp.dot(h, w2_ref[...], preferred_element_type=jnp.float32) + b2_ref[...]
    o_ref[...] = 1.0 / (1.0 + jnp.exp(-z))


def _mlp(u, q, time, W1, b1, W2, b2, interpret=False):
    RB = 2048
    grid = (B // RB,)
    W1u = W1[:EMB]
    W1q = W1[EMB:2 * EMB]
    w1t = W1[2 * EMB:]
    out = pl.pallas_call(
        _mlp_body,
        grid=grid,
        in_specs=[
            pl.BlockSpec((RB, EMB), lambda i: (i, 0)),
            pl.BlockSpec((RB, EMB), lambda i: (i, 0)),
            pl.BlockSpec((RB, 1), lambda i: (i, 0)),
            pl.BlockSpec((EMB, HID), lambda i: (0, 0)),
            pl.BlockSpec((EMB, HID), lambda i: (0, 0)),
            pl.BlockSpec((1, HID), lambda i: (0, 0)),
            pl.BlockSpec((1, HID), lambda i: (0, 0)),
            pl.BlockSpec((HID, 1), lambda i: (0, 0)),
            pl.BlockSpec((1, 1), lambda i: (0, 0)),
        ],
        out_specs=pl.BlockSpec((RB, 1), lambda i: (i, 0)),
        out_shape=jax.ShapeDtypeStruct((B, 1), jnp.float32),
        interpret=interpret,
    )(u, q, time, W1u, W1q, w1t, b1.reshape(1, HID), W2, b2.reshape(1, 1))
    return out.reshape(B)


def kernel(user, quiz, time, user_table, quiz_table, W1, b1, W2, b2):
    u, q = _sc_gather(user.astype(jnp.int32), quiz.astype(jnp.int32),
                      user_table, quiz_table)
    return _mlp(u, q, time, W1, b1, W2, b2)
